# baseline (speedup 1.0000x reference)
import functools

import jax
import jax.numpy as jnp
from jax import lax
from jax.experimental import pallas as pl
from jax.experimental.pallas import tpu as pltpu
from jax.experimental.pallas import tpu_sc as plsc

B = 1024
N = 1000
C = 128
K = 10
EPS = 1e-8

NUM_CORES = 2
NUM_SUBCORES = 16
NW = NUM_CORES * NUM_SUBCORES
ROWS = B * K
RPW = ROWS // NW
QPW = B // NW
CHUNK = 64
NCHUNK = RPW // CHUNK
LANES = 16


def _topk_body(x_ref, m_ref, inds_ref):
    x = x_ref[...]
    m = m_ref[...]
    dots = lax.dot_general(x, m, (((1,), (1,)), ((), ())),
                           preferred_element_type=jnp.float32)
    xn = jnp.sqrt(jnp.sum(x * x, axis=1, keepdims=True))
    mn = jnp.sqrt(jnp.sum(m * m, axis=1, keepdims=True))
    sims = dots / jnp.maximum(xn * mn.reshape(1, N), EPS)

    iota_n = lax.broadcasted_iota(jnp.int32, (B, N), 1)
    iota_cols = lax.broadcasted_iota(jnp.int32, (B, 128), 1)
    inds_acc = jnp.zeros((B, 128), jnp.int32)
    for j in range(K):
        rowmax = jnp.max(sims, axis=1, keepdims=True)
        cand = jnp.where(sims == rowmax, iota_n, jnp.int32(N))
        idx = jnp.min(cand, axis=1, keepdims=True)
        inds_acc = jnp.where(iota_cols == j, idx, inds_acc)
        sims = jnp.where(iota_n == idx, -jnp.inf, sims)
    inds_ref[...] = inds_acc


_topk = pl.pallas_call(
    _topk_body,
    out_shape=jax.ShapeDtypeStruct((B, 128), jnp.int32),
)


@functools.partial(
    pl.kernel,
    mesh=plsc.VectorSubcoreMesh(core_axis_name="c", subcore_axis_name="s"),
    out_type=jax.ShapeDtypeStruct((ROWS, C), jnp.float32),
    scratch_types=[
        pltpu.VMEM((RPW,), jnp.int32),
        pltpu.VMEM((RPW, C), jnp.float32),
        pltpu.SemaphoreType.DMA,
        pltpu.SemaphoreType.DMA,
    ],
)
def _gather(table_hbm, inds_hbm, out_hbm, idx_v, rows_v, gsem, wsem):
    wid = lax.axis_index("s") * NUM_CORES + lax.axis_index("c")
    pltpu.sync_copy(inds_hbm.at[pl.ds(wid * RPW, RPW)], idx_v)
    copies = []
    for c in range(NCHUNK):
        copies.append(pltpu.async_copy(
            table_hbm.at[idx_v.at[pl.ds(c * CHUNK, CHUNK)]],
            rows_v.at[pl.ds(c * CHUNK, CHUNK)],
            gsem,
        ))
    writes = []
    for c in range(NCHUNK):
        copies[c].wait()
        writes.append(pltpu.async_copy(
            rows_v.at[pl.ds(c * CHUNK, CHUNK)],
            out_hbm.at[pl.ds(wid * RPW + c * CHUNK, CHUNK)],
            wsem,
        ))
    for wr in writes:
        wr.wait()


def kernel(x, mod_embeddings, k):
    del k
    inds128 = _topk(x, mod_embeddings)
    flat = inds128[:, :K].reshape(ROWS)
    rows = _gather(mod_embeddings, flat)
    return rows.reshape(K, B, C).transpose(1, 0, 2)


# speedup vs baseline: 1.0731x; 1.0731x over previous
import functools

import jax
import jax.numpy as jnp
from jax import lax
from jax.experimental import pallas as pl
from jax.experimental.pallas import tpu as pltpu
from jax.experimental.pallas import tpu_sc as plsc

B = 1024
N = 1000
C = 128
K = 10
EPS = 1e-8

NUM_CORES = 2
NUM_SUBCORES = 16
NW = NUM_CORES * NUM_SUBCORES
ROWS = B * K
RPW = ROWS // NW
QPW = B // NW
CHUNK = 64
NCHUNK = RPW // CHUNK
LANES = 16


def _topk_body(x_ref, m_ref, inds_ref):
    x = x_ref[...]
    m = m_ref[...]
    dots = lax.dot_general(x, m, (((1,), (1,)), ((), ())),
                           preferred_element_type=jnp.float32)
    xn = jnp.sqrt(jnp.sum(x * x, axis=1, keepdims=True))
    mn = jnp.sqrt(jnp.sum(m * m, axis=1, keepdims=True))
    sims = dots / jnp.maximum(xn * mn.reshape(1, N), EPS)

    iota_f = lax.broadcasted_iota(jnp.int32, (B, N), 1).astype(jnp.float32)
    iota_cols = lax.broadcasted_iota(jnp.int32, (B, 128), 1)
    inds_acc = jnp.zeros((B, 128), jnp.int32)
    for j in range(K):
        rowmax = jnp.max(sims, axis=1, keepdims=True)
        cand = jnp.where(sims == rowmax, iota_f, jnp.float32(2048.0))
        idxf = jnp.min(cand, axis=1, keepdims=True)
        idx = idxf.astype(jnp.int32)
        inds_acc = jnp.where(iota_cols == j, idx, inds_acc)
        sims = jnp.where(iota_f == idxf, -jnp.inf, sims)
    inds_ref[...] = inds_acc


_topk = pl.pallas_call(
    _topk_body,
    out_shape=jax.ShapeDtypeStruct((B, 128), jnp.int32),
)


@functools.partial(
    pl.kernel,
    mesh=plsc.VectorSubcoreMesh(core_axis_name="c", subcore_axis_name="s"),
    out_type=jax.ShapeDtypeStruct((ROWS, C), jnp.float32),
    scratch_types=[
        pltpu.VMEM((RPW,), jnp.int32),
        pltpu.VMEM((RPW, C), jnp.float32),
        pltpu.SemaphoreType.DMA,
        pltpu.SemaphoreType.DMA,
    ],
)
def _gather(table_hbm, inds_hbm, out_hbm, idx_v, rows_v, gsem, wsem):
    wid = lax.axis_index("s") * NUM_CORES + lax.axis_index("c")
    pltpu.sync_copy(inds_hbm.at[pl.ds(wid * RPW, RPW)], idx_v)
    copies = []
    for c in range(NCHUNK):
        copies.append(pltpu.async_copy(
            table_hbm.at[idx_v.at[pl.ds(c * CHUNK, CHUNK)]],
            rows_v.at[pl.ds(c * CHUNK, CHUNK)],
            gsem,
        ))
    writes = []
    for c in range(NCHUNK):
        copies[c].wait()
        writes.append(pltpu.async_copy(
            rows_v.at[pl.ds(c * CHUNK, CHUNK)],
            out_hbm.at[pl.ds(wid * RPW + c * CHUNK, CHUNK)],
            wsem,
        ))
    for wr in writes:
        wr.wait()


def kernel(x, mod_embeddings, k):
    del k
    inds128 = _topk(x, mod_embeddings)
    flat = inds128[:, :K].reshape(ROWS)
    rows = _gather(mod_embeddings, flat)
    return rows.reshape(K, B, C).transpose(1, 0, 2)


# speedup vs baseline: 1.0743x; 1.0011x over previous
import functools

import jax
import jax.numpy as jnp
from jax import lax
from jax.experimental import pallas as pl
from jax.experimental.pallas import tpu as pltpu
from jax.experimental.pallas import tpu_sc as plsc

B = 1024
N = 1000
C = 128
K = 10
EPS = 1e-8

NUM_CORES = 2
NUM_SUBCORES = 16
NW = NUM_CORES * NUM_SUBCORES
ROWS = B * K
RPW = ROWS // NW
QPW = B // NW
CHUNKS = ((0, 112), (112, 112), (224, 96))
LANES = 16


def _topk_body(x_ref, m_ref, inds_ref):
    x = x_ref[...]
    m = m_ref[...]
    dots = lax.dot_general(x, m, (((1,), (1,)), ((), ())),
                           preferred_element_type=jnp.float32)
    xn = jnp.sqrt(jnp.sum(x * x, axis=1, keepdims=True))
    mn = jnp.sqrt(jnp.sum(m * m, axis=1, keepdims=True))
    sims = dots / jnp.maximum(xn * mn.reshape(1, N), EPS)

    iota_f = lax.broadcasted_iota(jnp.int32, (B, N), 1).astype(jnp.float32)
    iota_cols = lax.broadcasted_iota(jnp.int32, (B, 128), 1)
    inds_acc = jnp.zeros((B, 128), jnp.int32)
    for j in range(K):
        rowmax = jnp.max(sims, axis=1, keepdims=True)
        cand = jnp.where(sims == rowmax, iota_f, jnp.float32(2048.0))
        idxf = jnp.min(cand, axis=1, keepdims=True)
        idx = idxf.astype(jnp.int32)
        inds_acc = jnp.where(iota_cols == j, idx, inds_acc)
        sims = jnp.where(iota_f == idxf, -jnp.inf, sims)
    inds_ref[...] = inds_acc


_topk = pl.pallas_call(
    _topk_body,
    out_shape=jax.ShapeDtypeStruct((B, 128), jnp.int32),
)


@functools.partial(
    pl.kernel,
    mesh=plsc.VectorSubcoreMesh(core_axis_name="c", subcore_axis_name="s"),
    out_type=jax.ShapeDtypeStruct((ROWS, C), jnp.float32),
    scratch_types=[
        pltpu.VMEM((RPW,), jnp.int32),
        pltpu.VMEM((RPW, C), jnp.float32),
        pltpu.SemaphoreType.DMA,
        pltpu.SemaphoreType.DMA,
    ],
)
def _gather(table_hbm, inds_hbm, out_hbm, idx_v, rows_v, gsem, wsem):
    wid = lax.axis_index("s") * NUM_CORES + lax.axis_index("c")
    pltpu.sync_copy(inds_hbm.at[pl.ds(wid * RPW, RPW)], idx_v)
    copies = []
    for off, sz in CHUNKS:
        copies.append(pltpu.async_copy(
            table_hbm.at[idx_v.at[pl.ds(off, sz)]],
            rows_v.at[pl.ds(off, sz)],
            gsem,
        ))
    writes = []
    for i, (off, sz) in enumerate(CHUNKS):
        copies[i].wait()
        writes.append(pltpu.async_copy(
            rows_v.at[pl.ds(off, sz)],
            out_hbm.at[pl.ds(wid * RPW + off, sz)],
            wsem,
        ))
    for wr in writes:
        wr.wait()


def kernel(x, mod_embeddings, k):
    del k
    inds128 = _topk(x, mod_embeddings)
    flat = inds128[:, :K].reshape(ROWS)
    rows = _gather(mod_embeddings, flat)
    return rows.reshape(K, B, C).transpose(1, 0, 2)


# speedup vs baseline: 1.1667x; 1.0860x over previous
import functools

import jax
import jax.numpy as jnp
from jax import lax
from jax.experimental import pallas as pl
from jax.experimental.pallas import tpu as pltpu
from jax.experimental.pallas import tpu_sc as plsc

B = 1024
N = 1000
C = 128
K = 10
EPS = 1e-8

NUM_CORES = 2
NUM_SUBCORES = 16
NW = NUM_CORES * NUM_SUBCORES
ROWS = B * K
RPW = ROWS // NW
QPW = B // NW
CHUNKS = ((0, 112), (112, 112), (224, 96))
LANES = 16


def _topk_body(x_ref, m_ref, inds_ref):
    x = x_ref[...]
    m = m_ref[...]
    dots = lax.dot_general(x, m, (((1,), (1,)), ((), ())),
                           preferred_element_type=jnp.float32)
    xn = jnp.sqrt(jnp.sum(x * x, axis=1, keepdims=True))
    mn = jnp.sqrt(jnp.sum(m * m, axis=1, keepdims=True))
    sims = dots / jnp.maximum(xn * mn.reshape(1, N), EPS)
    sims = jnp.concatenate(
        [sims, jnp.full((B, 1024 - N), -jnp.inf, jnp.float32)], axis=1)

    iota_f = lax.broadcasted_iota(jnp.int32, (B, 1024), 1).astype(jnp.float32)
    iota_cols = lax.broadcasted_iota(jnp.int32, (B, 128), 1)
    inds_acc = jnp.zeros((B, 128), jnp.int32)
    for j in range(K):
        rowmax = jnp.max(sims, axis=1, keepdims=True)
        cand = jnp.where(sims == rowmax, iota_f, jnp.float32(2048.0))
        idxf = jnp.min(cand, axis=1, keepdims=True)
        idx = idxf.astype(jnp.int32)
        inds_acc = jnp.where(iota_cols == j, idx, inds_acc)
        sims = jnp.where(iota_f == idxf, -jnp.inf, sims)
    inds_ref[...] = inds_acc


_topk = pl.pallas_call(
    _topk_body,
    out_shape=jax.ShapeDtypeStruct((B, 128), jnp.int32),
)


@functools.partial(
    pl.kernel,
    mesh=plsc.VectorSubcoreMesh(core_axis_name="c", subcore_axis_name="s"),
    out_type=jax.ShapeDtypeStruct((ROWS, C), jnp.float32),
    scratch_types=[
        pltpu.VMEM((RPW,), jnp.int32),
        pltpu.VMEM((RPW, C), jnp.float32),
        pltpu.VMEM_SHARED((N, C), jnp.float32),
        pltpu.SemaphoreType.DMA,
        pltpu.SemaphoreType.DMA,
    ],
)
def _gather(table_hbm, inds_hbm, out_hbm, idx_v, rows_v, table_sp, gsem, wsem):
    wid = lax.axis_index("s") * NUM_CORES + lax.axis_index("c")
    sid = lax.axis_index("s")
    @pl.when(sid < 15)
    def _stage_full():
        pltpu.sync_copy(table_hbm.at[pl.ds(sid * 64, 64)],
                        table_sp.at[pl.ds(sid * 64, 64)])

    @pl.when(sid == 15)
    def _stage_tail():
        pltpu.sync_copy(table_hbm.at[pl.ds(960, 40)],
                        table_sp.at[pl.ds(960, 40)])
    pltpu.sync_copy(inds_hbm.at[pl.ds(wid * RPW, RPW)], idx_v)
    plsc.subcore_barrier()
    copies = []
    for off, sz in CHUNKS:
        copies.append(pltpu.async_copy(
            table_sp.at[idx_v.at[pl.ds(off, sz)]],
            rows_v.at[pl.ds(off, sz)],
            gsem,
        ))
    writes = []
    for i, (off, sz) in enumerate(CHUNKS):
        copies[i].wait()
        writes.append(pltpu.async_copy(
            rows_v.at[pl.ds(off, sz)],
            out_hbm.at[pl.ds(wid * RPW + off, sz)],
            wsem,
        ))
    for wr in writes:
        wr.wait()


def kernel(x, mod_embeddings, k):
    del k
    inds128 = _topk(x, mod_embeddings)
    flat = inds128[:, :K].reshape(ROWS)
    rows = _gather(mod_embeddings, flat)
    return rows.reshape(K, B, C).transpose(1, 0, 2)


# speedup vs baseline: 1.1684x; 1.0015x over previous
import functools

import jax
import jax.numpy as jnp
from jax import lax
from jax.experimental import pallas as pl
from jax.experimental.pallas import tpu as pltpu
from jax.experimental.pallas import tpu_sc as plsc

B = 1024
N = 1000
C = 128
K = 10
EPS = 1e-8

NUM_CORES = 2
NUM_SUBCORES = 16
NW = NUM_CORES * NUM_SUBCORES
ROWS = B * K
RPW = ROWS // NW
CHUNKS = ((0, 112), (112, 112), (224, 96))


def _topk_body(x_ref, m_ref, inds_ref):
    x = x_ref[...]
    m = m_ref[...]
    dots = lax.dot_general(x, m, (((1,), (1,)), ((), ())),
                           preferred_element_type=jnp.float32)
    xn = jnp.sqrt(jnp.sum(x * x, axis=1, keepdims=True))
    mn = jnp.sqrt(jnp.sum(m * m, axis=1, keepdims=True))
    sims = dots / jnp.maximum(xn * mn.reshape(1, N), EPS)
    sims = jnp.concatenate(
        [sims, jnp.full((B, 1024 - N), -jnp.inf, jnp.float32)], axis=1)

    iota_f = lax.broadcasted_iota(jnp.int32, (B, 1024), 1).astype(jnp.float32)
    iota_cols = lax.broadcasted_iota(jnp.int32, (B, 128), 1)
    inds_acc = jnp.zeros((B, 128), jnp.int32)
    for j in range(K):
        rowmax = jnp.max(sims, axis=1, keepdims=True)
        cand = jnp.where(sims == rowmax, iota_f, jnp.float32(2048.0))
        idxf = jnp.min(cand, axis=1, keepdims=True)
        idx = idxf.astype(jnp.int32)
        inds_acc = jnp.where(iota_cols == j, idx, inds_acc)
        sims = jnp.where(iota_f == idxf, -jnp.inf, sims)
    inds_ref[...] = inds_acc


_topk = pl.pallas_call(
    _topk_body,
    out_shape=jax.ShapeDtypeStruct((B, 128), jnp.int32),
)


@functools.partial(
    pl.kernel,
    mesh=plsc.VectorSubcoreMesh(core_axis_name="c", subcore_axis_name="s"),
    out_type=jax.ShapeDtypeStruct((ROWS, C), jnp.float32),
    scratch_types=[
        pltpu.VMEM((RPW,), jnp.int32),
        pltpu.VMEM((RPW, C), jnp.float32),
        pltpu.VMEM_SHARED((N, C), jnp.float32),
        pltpu.SemaphoreType.DMA,
        pltpu.SemaphoreType.DMA,
    ],
)
def _gather(table_hbm, inds_hbm, out_hbm, idx_v, rows_v, table_sp, gsem, wsem):
    wid = lax.axis_index("s") * NUM_CORES + lax.axis_index("c")
    sid = lax.axis_index("s")
    @pl.when(sid < 15)
    def _stage_full():
        pltpu.sync_copy(table_hbm.at[pl.ds(sid * 64, 64)],
                        table_sp.at[pl.ds(sid * 64, 64)])

    @pl.when(sid == 15)
    def _stage_tail():
        pltpu.sync_copy(table_hbm.at[pl.ds(960, 40)],
                        table_sp.at[pl.ds(960, 40)])
    pltpu.sync_copy(inds_hbm.at[pl.ds(wid * RPW, RPW)], idx_v)
    plsc.subcore_barrier()
    copies = []
    for off, sz in CHUNKS:
        copies.append(pltpu.async_copy(
            table_sp.at[idx_v.at[pl.ds(off, sz)]],
            rows_v.at[pl.ds(off, sz)],
            gsem,
        ))
    writes = []
    for i, (off, sz) in enumerate(CHUNKS):
        copies[i].wait()
        writes.append(pltpu.async_copy(
            rows_v.at[pl.ds(off, sz)],
            out_hbm.at[pl.ds(wid * RPW + off, sz)],
            wsem,
        ))
    for wr in writes:
        wr.wait()


def kernel(x, mod_embeddings, k):
    del k
    inds128 = _topk(x, mod_embeddings)
    flat = inds128[:, :K].reshape(ROWS)
    rows = _gather(mod_embeddings, flat)
    return rows.reshape(K, B, C).transpose(1, 0, 2)
